# baseline (speedup 1.0000x reference)
import functools

import jax
import jax.numpy as jnp
from jax import lax
from jax.experimental import pallas as pl
from jax.experimental.pallas import tpu as pltpu
from jax.experimental.pallas import tpu_sc as plsc

D = 512
B = 4096

_NEG_INF = float('-inf')
K_MEX = 24
_CH = 512
_NCH = B // _CH

_f32 = jnp.float32
_i32 = jnp.int32

_GCH = 16
_NCK = 8
_NSLOT = 4


def _make_sc_gather2():
    info = plsc.get_sparse_core_info()
    nc, ns = info.num_cores, info.num_subcores
    nw = nc * ns
    bpw = B // nw
    assert bpw == _GCH * _NCK

    mesh = plsc.VectorSubcoreMesh(core_axis_name="c", subcore_axis_name="s")

    @functools.partial(
        pl.kernel,
        mesh=mesh,
        out_type=[
            jax.ShapeDtypeStruct((B, D), _f32),
            jax.ShapeDtypeStruct((B, D), _f32),
        ],
        scratch_types=(
            [pltpu.VMEM((_GCH, D), _f32)] * (2 * _NSLOT)
            + [pltpu.VMEM((_GCH,), _i32)] * (2 * _NSLOT)
            + [pltpu.SemaphoreType.DMA] * (2 * _NSLOT)
        ),
    )
    def gather2(tab_a, tab_b, idx_a, idx_b, out_a, out_b, *scr):
        bufs = scr[0:2 * _NSLOT]
        ibufs = scr[2 * _NSLOT:4 * _NSLOT]
        sems = scr[4 * _NSLOT:6 * _NSLOT]
        tabs = (tab_a, tab_b)
        idxs = (idx_a, idx_b)
        outs = (out_a, out_b)
        wid = lax.axis_index("s") * nc + lax.axis_index("c")
        base = wid * bpw
        handles = {}

        def fire(t, c):
            slot = _NSLOT * t + (c % _NSLOT)
            pltpu.sync_copy(idxs[t].at[pl.ds(base + c * _GCH, _GCH)],
                            ibufs[slot])
            handles[(t, c)] = pltpu.async_copy(
                tabs[t].at[ibufs[slot]], bufs[slot], sems[slot])

        def drain(t, c):
            slot = _NSLOT * t + (c % _NSLOT)
            handles[(t, c)].wait()
            pltpu.sync_copy(bufs[slot],
                            outs[t].at[pl.ds(base + c * _GCH, _GCH)])

        for c in range(_NSLOT):
            fire(0, c)
            fire(1, c)
        for c in range(_NCK):
            drain(0, c)
            if c + _NSLOT < _NCK:
                fire(0, c + _NSLOT)
            drain(1, c)
            if c + _NSLOT < _NCK:
                fire(1, c + _NSLOT)

    return gather2


def _dense_body(mi_emb, di_emb, w_mi, b_mi, w_di, b_di, mi_h_o, di_h_o, r_o):
    mi_h = jnp.dot(mi_emb[...], w_mi[...], preferred_element_type=_f32)
    mi_h = mi_h + b_mi[...]
    di_h = jnp.dot(di_emb[...], w_di[...], preferred_element_type=_f32)
    di_h = di_h + b_di[...]
    num = jnp.sum(mi_h * di_h, axis=1, keepdims=True)
    n1 = jnp.sqrt(jnp.sum(mi_h * mi_h, axis=1, keepdims=True))
    n2 = jnp.sqrt(jnp.sum(di_h * di_h, axis=1, keepdims=True))
    mi_h_o[...] = mi_h
    di_h_o[...] = di_h
    r_o[...] = num / (n1 * n2)


def _dense_call(mi_emb, di_emb, w_mi, b_mi, w_di, b_di, interpret=False):
    return pl.pallas_call(
        _dense_body,
        out_shape=[
            jax.ShapeDtypeStruct((B, D), _f32),
            jax.ShapeDtypeStruct((B, D), _f32),
            jax.ShapeDtypeStruct((B, 1), _f32),
        ],
        interpret=interpret,
    )(mi_emb, di_emb, w_mi, b_mi.reshape(1, D), w_di, b_di.reshape(1, D))


def _first_body(mi_r, di_r, mi_c, di_c, fmi_o, fdi_o, pfirst_o):
    j0 = pl.program_id(0) * _CH
    jp = lax.broadcasted_iota(_i32, (1, _CH), 1) + j0
    iota_c = lax.broadcasted_iota(_i32, (B, 1), 0)
    klt = iota_c < jp
    eq_mi = mi_c[...] == mi_r[...]
    eq_di = di_c[...] == di_r[...]
    cnt_mi = jnp.sum((eq_mi & klt).astype(_f32), axis=0, keepdims=True)
    cnt_di = jnp.sum((eq_di & klt).astype(_f32), axis=0, keepdims=True)
    cnt_pr = jnp.sum((eq_mi & eq_di & klt).astype(_f32), axis=0, keepdims=True)
    fmi_o[...] = (cnt_mi == 0.0).astype(_f32)
    fdi_o[...] = (cnt_di == 0.0).astype(_f32)
    pfirst_o[...] = (cnt_pr == 0.0).astype(_f32)


def _first_call(mi_c, mi_r, di_c, di_r, interpret=False):
    return pl.pallas_call(
        _first_body,
        grid=(_NCH,),
        in_specs=[
            pl.BlockSpec((1, _CH), lambda j: (0, j)),
            pl.BlockSpec((1, _CH), lambda j: (0, j)),
            pl.BlockSpec((B, 1), lambda j: (0, 0)),
            pl.BlockSpec((B, 1), lambda j: (0, 0)),
        ],
        out_specs=[
            pl.BlockSpec((1, _CH), lambda j: (0, j)),
            pl.BlockSpec((1, _CH), lambda j: (0, j)),
            pl.BlockSpec((1, _CH), lambda j: (0, j)),
        ],
        out_shape=[
            jax.ShapeDtypeStruct((1, B), _f32),
            jax.ShapeDtypeStruct((1, B), _f32),
            jax.ShapeDtypeStruct((1, B), _f32),
        ],
        interpret=interpret,
    )(mi_r, di_r, mi_c, di_c)


def _crank_body(mi_i, di_i, mi_r, di_r, fmi_r, fdi_r, crmi_o, crdi_o):
    crmi_o[...] = jnp.sum(jnp.where(mi_r[...] < mi_i[...], fmi_r[...], 0.0),
                          axis=1, keepdims=True)
    crdi_o[...] = jnp.sum(jnp.where(di_r[...] < di_i[...], fdi_r[...], 0.0),
                          axis=1, keepdims=True)


def _crank_call(mi_c, mi_r, di_c, di_r, fmi_r, fdi_r, interpret=False):
    return pl.pallas_call(
        _crank_body,
        grid=(_NCH,),
        in_specs=[
            pl.BlockSpec((_CH, 1), lambda j: (j, 0)),
            pl.BlockSpec((_CH, 1), lambda j: (j, 0)),
            pl.BlockSpec((1, B), lambda j: (0, 0)),
            pl.BlockSpec((1, B), lambda j: (0, 0)),
            pl.BlockSpec((1, B), lambda j: (0, 0)),
            pl.BlockSpec((1, B), lambda j: (0, 0)),
        ],
        out_specs=[
            pl.BlockSpec((_CH, 1), lambda j: (j, 0)),
            pl.BlockSpec((_CH, 1), lambda j: (j, 0)),
        ],
        out_shape=[
            jax.ShapeDtypeStruct((B, 1), _f32),
            jax.ShapeDtypeStruct((B, 1), _f32),
        ],
        interpret=interpret,
    )(mi_c, di_c, mi_r, di_r, fmi_r, fdi_r)


def _one_greedy(row_cv, row_chunk, r_cv, crank, pw, n_col):
    same = row_cv == row_chunk
    wmask = jnp.where(same, r_cv, _NEG_INF)
    m = jnp.max(wmask, axis=0, keepdims=True)
    ach = same & (r_cv == m)
    bc = jnp.min(jnp.where(ach, crank, 1e9), axis=0, keepdims=True)
    bits = jnp.sum(jnp.where(same, pw, 0.0), axis=0, keepdims=True)
    x = bits.astype(_i32)
    y = jnp.bitwise_and(jnp.bitwise_not(x), (1 << K_MEX) - 1)
    lsb = jnp.bitwise_and(y, -y)
    mex = (lax.shift_right_logical(
        lax.bitcast_convert_type(lsb.astype(_f32), _i32), 23) - 127)
    mex_f = mex.astype(_f32)
    use_mex = (y != 0) & (mex_f < n_col) & (m <= 0.0)
    return jnp.where(use_mex, mex_f, bc).astype(_i32)


def _greedy_body(mi_c, di_c, mi_r, di_r, r_c, crmi_c, crdi_c, pfirst_c,
                 fmi_r, fdi_r, res_mi_o, res_di_o):
    n_col_mi = jnp.sum(fmi_r[...])
    n_col_di = jnp.sum(fdi_r[...])
    r_cv = r_c[...]
    pf = pfirst_c[...] > 0.0

    def pow2_of(crank):
        crank_i = crank.astype(_i32)
        p = lax.bitcast_convert_type(lax.shift_left(crank_i + 127, 23), _f32)
        return jnp.where(pf & (crank < float(K_MEX)), p, 0.0)

    crdi = crdi_c[...]
    crmi = crmi_c[...]
    res_mi_o[...] = _one_greedy(mi_c[...], mi_r[...], r_cv, crdi,
                                pow2_of(crdi), n_col_di)
    res_di_o[...] = _one_greedy(di_c[...], di_r[...], r_cv, crmi,
                                pow2_of(crmi), n_col_mi)


def _greedy_call(mi_c, mi_r, di_c, di_r, r_c, crmi_c, crdi_c, pfirst_c,
                 fmi_r, fdi_r, interpret=False):
    return pl.pallas_call(
        _greedy_body,
        grid=(_NCH,),
        in_specs=[
            pl.BlockSpec((B, 1), lambda j: (0, 0)),
            pl.BlockSpec((B, 1), lambda j: (0, 0)),
            pl.BlockSpec((1, _CH), lambda j: (0, j)),
            pl.BlockSpec((1, _CH), lambda j: (0, j)),
            pl.BlockSpec((B, 1), lambda j: (0, 0)),
            pl.BlockSpec((B, 1), lambda j: (0, 0)),
            pl.BlockSpec((B, 1), lambda j: (0, 0)),
            pl.BlockSpec((B, 1), lambda j: (0, 0)),
            pl.BlockSpec((1, B), lambda j: (0, 0)),
            pl.BlockSpec((1, B), lambda j: (0, 0)),
        ],
        out_specs=[
            pl.BlockSpec((1, _CH), lambda j: (0, j)),
            pl.BlockSpec((1, _CH), lambda j: (0, j)),
        ],
        out_shape=[
            jax.ShapeDtypeStruct((1, B), _i32),
            jax.ShapeDtypeStruct((1, B), _i32),
        ],
        interpret=interpret,
    )(mi_c, di_c, mi_r, di_r, r_c, crmi_c, crdi_c, pfirst_c, fmi_r, fdi_r)


def _best_loss_body(res_mi_c, res_di_c, mi_hj, di_hj, mi_hf, di_hf,
                    best_mi_o, best_di_o, loss_o):
    j = pl.program_id(0)

    @pl.when(j < _NCH)
    def _gather_step():
        j0 = j * _CH
        jpos = lax.broadcasted_iota(_i32, (1, _CH), 1) + j0
        pmi = (res_mi_c[...] == jpos).astype(jnp.bfloat16)
        pdi = (res_di_c[...] == jpos).astype(jnp.bfloat16)
        cmi = jnp.dot(pmi, mi_hj[...].astype(jnp.bfloat16),
                      preferred_element_type=_f32)
        cdi = jnp.dot(pdi, di_hj[...].astype(jnp.bfloat16),
                      preferred_element_type=_f32)

        @pl.when(j == 0)
        def _init():
            best_mi_o[...] = cmi
            best_di_o[...] = cdi

        @pl.when(j > 0)
        def _acc():
            best_mi_o[...] = best_mi_o[...] + cmi
            best_di_o[...] = best_di_o[...] + cdi

    @pl.when(j == _NCH)
    def _loss_step():
        d1 = mi_hf[...] - best_mi_o[...]
        d2 = di_hf[...] - best_di_o[...]
        s = jnp.sum(d1 * d1) + jnp.sum(d2 * d2)
        loss_o[...] = jnp.broadcast_to(s / (2.0 * B * D), (1, 1))


def _best_loss_call(res_mi_c, res_di_c, mi_h, di_h, interpret=False):
    return pl.pallas_call(
        _best_loss_body,
        grid=(_NCH + 1,),
        in_specs=[
            pl.BlockSpec((B, 1), lambda j: (0, 0)),
            pl.BlockSpec((B, 1), lambda j: (0, 0)),
            pl.BlockSpec((_CH, D), lambda j: (jnp.minimum(j, _NCH - 1), 0)),
            pl.BlockSpec((_CH, D), lambda j: (jnp.minimum(j, _NCH - 1), 0)),
            pl.BlockSpec((B, D), lambda j: (0, 0)),
            pl.BlockSpec((B, D), lambda j: (0, 0)),
        ],
        out_specs=[
            pl.BlockSpec((B, D), lambda j: (0, 0)),
            pl.BlockSpec((B, D), lambda j: (0, 0)),
            pl.BlockSpec((1, 1), lambda j: (0, 0)),
        ],
        out_shape=[
            jax.ShapeDtypeStruct((B, D), _f32),
            jax.ShapeDtypeStruct((B, D), _f32),
            jax.ShapeDtypeStruct((1, 1), _f32),
        ],
        interpret=interpret,
    )(res_mi_c, res_di_c, mi_h, di_h, mi_h, di_h)


def kernel(miRNA_embeddings, disease_embeddings, W_mi, b_mi, W_di, b_di,
           miRNA_index, disease_index):
    mi_c = miRNA_index.reshape(B, 1)
    mi_r = miRNA_index.reshape(1, B)
    di_c = disease_index.reshape(B, 1)
    di_r = disease_index.reshape(1, B)
    fmi_r, fdi_r, pfirst_r = _first_call(mi_c, mi_r, di_c, di_r)
    crmi_c, crdi_c = _crank_call(mi_c, mi_r, di_c, di_r, fmi_r, fdi_r)
    gather2 = _make_sc_gather2()
    mi_emb, di_emb = gather2(miRNA_embeddings, disease_embeddings,
                             miRNA_index, disease_index)
    mi_h, di_h, r_col = _dense_call(mi_emb, di_emb, W_mi, b_mi, W_di, b_di)
    res_mi, res_di = _greedy_call(mi_c, mi_r, di_c, di_r, r_col,
                                  crmi_c, crdi_c, pfirst_r.reshape(B, 1),
                                  fmi_r, fdi_r)
    best_mi, best_di, nash_loss = _best_loss_call(
        res_mi.reshape(B, 1), res_di.reshape(B, 1), mi_h, di_h)
    return (nash_loss.reshape(()), best_mi, best_di)


# speedup vs baseline: 1.0011x; 1.0011x over previous
import functools

import jax
import jax.numpy as jnp
from jax import lax
from jax.experimental import pallas as pl
from jax.experimental.pallas import tpu as pltpu
from jax.experimental.pallas import tpu_sc as plsc

D = 512
B = 4096

_NEG_INF = float('-inf')
K_MEX = 24
_CH = 512
_NCH = B // _CH

_f32 = jnp.float32
_i32 = jnp.int32

_GCH = 16
_NCK = 8
_NSLOT = 6


def _make_sc_gather2():
    info = plsc.get_sparse_core_info()
    nc, ns = info.num_cores, info.num_subcores
    nw = nc * ns
    bpw = B // nw
    assert bpw == _GCH * _NCK

    mesh = plsc.VectorSubcoreMesh(core_axis_name="c", subcore_axis_name="s")

    @functools.partial(
        pl.kernel,
        mesh=mesh,
        out_type=[
            jax.ShapeDtypeStruct((B, D), _f32),
            jax.ShapeDtypeStruct((B, D), _f32),
        ],
        scratch_types=(
            [pltpu.VMEM((_GCH, D), _f32)] * (2 * _NSLOT)
            + [pltpu.VMEM((_GCH,), _i32)] * (2 * _NSLOT)
            + [pltpu.SemaphoreType.DMA] * (2 * _NSLOT)
        ),
    )
    def gather2(tab_a, tab_b, idx_a, idx_b, out_a, out_b, *scr):
        bufs = scr[0:2 * _NSLOT]
        ibufs = scr[2 * _NSLOT:4 * _NSLOT]
        sems = scr[4 * _NSLOT:6 * _NSLOT]
        tabs = (tab_a, tab_b)
        idxs = (idx_a, idx_b)
        outs = (out_a, out_b)
        wid = lax.axis_index("s") * nc + lax.axis_index("c")
        base = wid * bpw
        handles = {}

        def fire(t, c):
            slot = _NSLOT * t + (c % _NSLOT)
            pltpu.sync_copy(idxs[t].at[pl.ds(base + c * _GCH, _GCH)],
                            ibufs[slot])
            handles[(t, c)] = pltpu.async_copy(
                tabs[t].at[ibufs[slot]], bufs[slot], sems[slot])

        def drain(t, c):
            slot = _NSLOT * t + (c % _NSLOT)
            handles[(t, c)].wait()
            pltpu.sync_copy(bufs[slot],
                            outs[t].at[pl.ds(base + c * _GCH, _GCH)])

        for c in range(_NSLOT):
            fire(0, c)
            fire(1, c)
        for c in range(_NCK):
            drain(0, c)
            if c + _NSLOT < _NCK:
                fire(0, c + _NSLOT)
            drain(1, c)
            if c + _NSLOT < _NCK:
                fire(1, c + _NSLOT)

    return gather2


def _dense_body(mi_emb, di_emb, w_mi, b_mi, w_di, b_di, mi_h_o, di_h_o, r_o):
    mi_h = jnp.dot(mi_emb[...], w_mi[...], preferred_element_type=_f32)
    mi_h = mi_h + b_mi[...]
    di_h = jnp.dot(di_emb[...], w_di[...], preferred_element_type=_f32)
    di_h = di_h + b_di[...]
    num = jnp.sum(mi_h * di_h, axis=1, keepdims=True)
    n1 = jnp.sqrt(jnp.sum(mi_h * mi_h, axis=1, keepdims=True))
    n2 = jnp.sqrt(jnp.sum(di_h * di_h, axis=1, keepdims=True))
    mi_h_o[...] = mi_h
    di_h_o[...] = di_h
    r_o[...] = num / (n1 * n2)


def _dense_call(mi_emb, di_emb, w_mi, b_mi, w_di, b_di, interpret=False):
    return pl.pallas_call(
        _dense_body,
        out_shape=[
            jax.ShapeDtypeStruct((B, D), _f32),
            jax.ShapeDtypeStruct((B, D), _f32),
            jax.ShapeDtypeStruct((B, 1), _f32),
        ],
        interpret=interpret,
    )(mi_emb, di_emb, w_mi, b_mi.reshape(1, D), w_di, b_di.reshape(1, D))


def _first_body(mi_r, di_r, mi_c, di_c, fmi_o, fdi_o, pfirst_o):
    j0 = pl.program_id(0) * _CH
    jp = lax.broadcasted_iota(_i32, (1, _CH), 1) + j0
    iota_c = lax.broadcasted_iota(_i32, (B, 1), 0)
    klt = iota_c < jp
    eq_mi = mi_c[...] == mi_r[...]
    eq_di = di_c[...] == di_r[...]
    cnt_mi = jnp.sum((eq_mi & klt).astype(_f32), axis=0, keepdims=True)
    cnt_di = jnp.sum((eq_di & klt).astype(_f32), axis=0, keepdims=True)
    cnt_pr = jnp.sum((eq_mi & eq_di & klt).astype(_f32), axis=0, keepdims=True)
    fmi_o[...] = (cnt_mi == 0.0).astype(_f32)
    fdi_o[...] = (cnt_di == 0.0).astype(_f32)
    pfirst_o[...] = (cnt_pr == 0.0).astype(_f32)


def _first_call(mi_c, mi_r, di_c, di_r, interpret=False):
    return pl.pallas_call(
        _first_body,
        grid=(_NCH,),
        in_specs=[
            pl.BlockSpec((1, _CH), lambda j: (0, j)),
            pl.BlockSpec((1, _CH), lambda j: (0, j)),
            pl.BlockSpec((B, 1), lambda j: (0, 0)),
            pl.BlockSpec((B, 1), lambda j: (0, 0)),
        ],
        out_specs=[
            pl.BlockSpec((1, _CH), lambda j: (0, j)),
            pl.BlockSpec((1, _CH), lambda j: (0, j)),
            pl.BlockSpec((1, _CH), lambda j: (0, j)),
        ],
        out_shape=[
            jax.ShapeDtypeStruct((1, B), _f32),
            jax.ShapeDtypeStruct((1, B), _f32),
            jax.ShapeDtypeStruct((1, B), _f32),
        ],
        interpret=interpret,
    )(mi_r, di_r, mi_c, di_c)


def _crank_body(mi_i, di_i, mi_r, di_r, fmi_r, fdi_r, crmi_o, crdi_o):
    crmi_o[...] = jnp.sum(jnp.where(mi_r[...] < mi_i[...], fmi_r[...], 0.0),
                          axis=1, keepdims=True)
    crdi_o[...] = jnp.sum(jnp.where(di_r[...] < di_i[...], fdi_r[...], 0.0),
                          axis=1, keepdims=True)


def _crank_call(mi_c, mi_r, di_c, di_r, fmi_r, fdi_r, interpret=False):
    return pl.pallas_call(
        _crank_body,
        grid=(_NCH,),
        in_specs=[
            pl.BlockSpec((_CH, 1), lambda j: (j, 0)),
            pl.BlockSpec((_CH, 1), lambda j: (j, 0)),
            pl.BlockSpec((1, B), lambda j: (0, 0)),
            pl.BlockSpec((1, B), lambda j: (0, 0)),
            pl.BlockSpec((1, B), lambda j: (0, 0)),
            pl.BlockSpec((1, B), lambda j: (0, 0)),
        ],
        out_specs=[
            pl.BlockSpec((_CH, 1), lambda j: (j, 0)),
            pl.BlockSpec((_CH, 1), lambda j: (j, 0)),
        ],
        out_shape=[
            jax.ShapeDtypeStruct((B, 1), _f32),
            jax.ShapeDtypeStruct((B, 1), _f32),
        ],
        interpret=interpret,
    )(mi_c, di_c, mi_r, di_r, fmi_r, fdi_r)


def _one_greedy(row_cv, row_chunk, r_cv, crank, pw, n_col):
    same = row_cv == row_chunk
    wmask = jnp.where(same, r_cv, _NEG_INF)
    m = jnp.max(wmask, axis=0, keepdims=True)
    ach = same & (r_cv == m)
    bc = jnp.min(jnp.where(ach, crank, 1e9), axis=0, keepdims=True)
    bits = jnp.sum(jnp.where(same, pw, 0.0), axis=0, keepdims=True)
    x = bits.astype(_i32)
    y = jnp.bitwise_and(jnp.bitwise_not(x), (1 << K_MEX) - 1)
    lsb = jnp.bitwise_and(y, -y)
    mex = (lax.shift_right_logical(
        lax.bitcast_convert_type(lsb.astype(_f32), _i32), 23) - 127)
    mex_f = mex.astype(_f32)
    use_mex = (y != 0) & (mex_f < n_col) & (m <= 0.0)
    return jnp.where(use_mex, mex_f, bc).astype(_i32)


def _greedy_body(mi_c, di_c, mi_r, di_r, r_c, crmi_c, crdi_c, pfirst_c,
                 fmi_r, fdi_r, res_mi_o, res_di_o):
    n_col_mi = jnp.sum(fmi_r[...])
    n_col_di = jnp.sum(fdi_r[...])
    r_cv = r_c[...]
    pf = pfirst_c[...] > 0.0

    def pow2_of(crank):
        crank_i = crank.astype(_i32)
        p = lax.bitcast_convert_type(lax.shift_left(crank_i + 127, 23), _f32)
        return jnp.where(pf & (crank < float(K_MEX)), p, 0.0)

    crdi = crdi_c[...]
    crmi = crmi_c[...]
    res_mi_o[...] = _one_greedy(mi_c[...], mi_r[...], r_cv, crdi,
                                pow2_of(crdi), n_col_di)
    res_di_o[...] = _one_greedy(di_c[...], di_r[...], r_cv, crmi,
                                pow2_of(crmi), n_col_mi)


def _greedy_call(mi_c, mi_r, di_c, di_r, r_c, crmi_c, crdi_c, pfirst_c,
                 fmi_r, fdi_r, interpret=False):
    return pl.pallas_call(
        _greedy_body,
        grid=(_NCH,),
        in_specs=[
            pl.BlockSpec((B, 1), lambda j: (0, 0)),
            pl.BlockSpec((B, 1), lambda j: (0, 0)),
            pl.BlockSpec((1, _CH), lambda j: (0, j)),
            pl.BlockSpec((1, _CH), lambda j: (0, j)),
            pl.BlockSpec((B, 1), lambda j: (0, 0)),
            pl.BlockSpec((B, 1), lambda j: (0, 0)),
            pl.BlockSpec((B, 1), lambda j: (0, 0)),
            pl.BlockSpec((B, 1), lambda j: (0, 0)),
            pl.BlockSpec((1, B), lambda j: (0, 0)),
            pl.BlockSpec((1, B), lambda j: (0, 0)),
        ],
        out_specs=[
            pl.BlockSpec((1, _CH), lambda j: (0, j)),
            pl.BlockSpec((1, _CH), lambda j: (0, j)),
        ],
        out_shape=[
            jax.ShapeDtypeStruct((1, B), _i32),
            jax.ShapeDtypeStruct((1, B), _i32),
        ],
        interpret=interpret,
    )(mi_c, di_c, mi_r, di_r, r_c, crmi_c, crdi_c, pfirst_c, fmi_r, fdi_r)


def _best_loss_body(res_mi_c, res_di_c, mi_hj, di_hj, mi_hf, di_hf,
                    best_mi_o, best_di_o, loss_o):
    j = pl.program_id(0)

    @pl.when(j < _NCH)
    def _gather_step():
        j0 = j * _CH
        jpos = lax.broadcasted_iota(_i32, (1, _CH), 1) + j0
        pmi = (res_mi_c[...] == jpos).astype(jnp.bfloat16)
        pdi = (res_di_c[...] == jpos).astype(jnp.bfloat16)
        cmi = jnp.dot(pmi, mi_hj[...].astype(jnp.bfloat16),
                      preferred_element_type=_f32)
        cdi = jnp.dot(pdi, di_hj[...].astype(jnp.bfloat16),
                      preferred_element_type=_f32)

        @pl.when(j == 0)
        def _init():
            best_mi_o[...] = cmi
            best_di_o[...] = cdi

        @pl.when(j > 0)
        def _acc():
            best_mi_o[...] = best_mi_o[...] + cmi
            best_di_o[...] = best_di_o[...] + cdi

    @pl.when(j == _NCH)
    def _loss_step():
        d1 = mi_hf[...] - best_mi_o[...]
        d2 = di_hf[...] - best_di_o[...]
        s = jnp.sum(d1 * d1) + jnp.sum(d2 * d2)
        loss_o[...] = jnp.broadcast_to(s / (2.0 * B * D), (1, 1))


def _best_loss_call(res_mi_c, res_di_c, mi_h, di_h, interpret=False):
    return pl.pallas_call(
        _best_loss_body,
        grid=(_NCH + 1,),
        in_specs=[
            pl.BlockSpec((B, 1), lambda j: (0, 0)),
            pl.BlockSpec((B, 1), lambda j: (0, 0)),
            pl.BlockSpec((_CH, D), lambda j: (jnp.minimum(j, _NCH - 1), 0)),
            pl.BlockSpec((_CH, D), lambda j: (jnp.minimum(j, _NCH - 1), 0)),
            pl.BlockSpec((B, D), lambda j: (0, 0)),
            pl.BlockSpec((B, D), lambda j: (0, 0)),
        ],
        out_specs=[
            pl.BlockSpec((B, D), lambda j: (0, 0)),
            pl.BlockSpec((B, D), lambda j: (0, 0)),
            pl.BlockSpec((1, 1), lambda j: (0, 0)),
        ],
        out_shape=[
            jax.ShapeDtypeStruct((B, D), _f32),
            jax.ShapeDtypeStruct((B, D), _f32),
            jax.ShapeDtypeStruct((1, 1), _f32),
        ],
        interpret=interpret,
    )(res_mi_c, res_di_c, mi_h, di_h, mi_h, di_h)


def kernel(miRNA_embeddings, disease_embeddings, W_mi, b_mi, W_di, b_di,
           miRNA_index, disease_index):
    mi_c = miRNA_index.reshape(B, 1)
    mi_r = miRNA_index.reshape(1, B)
    di_c = disease_index.reshape(B, 1)
    di_r = disease_index.reshape(1, B)
    fmi_r, fdi_r, pfirst_r = _first_call(mi_c, mi_r, di_c, di_r)
    crmi_c, crdi_c = _crank_call(mi_c, mi_r, di_c, di_r, fmi_r, fdi_r)
    gather2 = _make_sc_gather2()
    mi_emb, di_emb = gather2(miRNA_embeddings, disease_embeddings,
                             miRNA_index, disease_index)
    mi_h, di_h, r_col = _dense_call(mi_emb, di_emb, W_mi, b_mi, W_di, b_di)
    res_mi, res_di = _greedy_call(mi_c, mi_r, di_c, di_r, r_col,
                                  crmi_c, crdi_c, pfirst_r.reshape(B, 1),
                                  fmi_r, fdi_r)
    best_mi, best_di, nash_loss = _best_loss_call(
        res_mi.reshape(B, 1), res_di.reshape(B, 1), mi_h, di_h)
    return (nash_loss.reshape(()), best_mi, best_di)
